# Initial kernel scaffold; baseline (speedup 1.0000x reference)
import jax
import jax.numpy as jnp
from jax.experimental import pallas as pl


def kernel(x, subspace_params, centroids):
    raise NotImplementedError("write your pallas kernel here")


# speedup vs baseline: 1.7552x; 1.7552x over previous
import jax
import jax.numpy as jnp
from jax.experimental import pallas as pl

D_MODEL = 2048
K_CODES = 512
CODE_DIM = 64
TAU = 1.0

_BG = 2048
_BM = 512
_BN = 2048


def _quantize_block(z_ref, c_ref, q_ref):
    z = z_ref[...]
    c = c_ref[...]
    c2 = jnp.sum(c * c, axis=1)[None, :]
    logits = (2.0 * jnp.dot(z, c.T, preferred_element_type=jnp.float32)
              - c2) * (1.0 / TAU)
    m = jnp.max(logits, axis=1, keepdims=True)
    e = jnp.exp(logits - m)
    s = jnp.sum(e, axis=1, keepdims=True)
    q_ref[...] = jnp.dot(e, c, preferred_element_type=jnp.float32) / s


def _matmul_block(x_ref, w_ref, o_ref):
    o_ref[...] = jnp.dot(x_ref[...], w_ref[...],
                         preferred_element_type=jnp.float32)


def kernel(x, subspace_params, centroids):
    z = subspace_params.reshape(-1, CODE_DIM)
    g = z.shape[0]

    q = pl.pallas_call(
        _quantize_block,
        grid=(g // _BG,),
        in_specs=[
            pl.BlockSpec((_BG, CODE_DIM), lambda i: (i, 0)),
            pl.BlockSpec((K_CODES, CODE_DIM), lambda i: (0, 0)),
        ],
        out_specs=pl.BlockSpec((_BG, CODE_DIM), lambda i: (i, 0)),
        out_shape=jax.ShapeDtypeStruct((g, CODE_DIM), jnp.float32),
    )(z, centroids)

    w = q.reshape(D_MODEL, D_MODEL)

    m = x.shape[0]
    out = pl.pallas_call(
        _matmul_block,
        grid=(m // _BM, D_MODEL // _BN),
        in_specs=[
            pl.BlockSpec((_BM, D_MODEL), lambda i, j: (i, 0)),
            pl.BlockSpec((D_MODEL, _BN), lambda i, j: (0, j)),
        ],
        out_specs=pl.BlockSpec((_BM, _BN), lambda i, j: (i, j)),
        out_shape=jax.ShapeDtypeStruct((m, D_MODEL), jnp.float32),
    )(x, w)
    return out
